# baseline (speedup 1.0000x reference)
import jax
import jax.numpy as jnp
from jax.experimental import pallas as pl
from jax.experimental.pallas import tpu as pltpu

C = 128
G = 8
CG = 16
EPS = 1e-5
WP = 24
MRG = 8
NL = 4
BF = jnp.bfloat16


def _make_tower_kernel(D, W, Pp, co):
    D2 = D + 2
    M = D * Pp
    ob = MRG + Pp + WP
    cnt = float(D * W * W * CG)

    def kern(x_ref, w_ref, b_ref, g_ref, be_ref, wo_ref, bo_ref, o_ref,
             im_ref, pb_ref):
        o_ref[...] = jnp.zeros_like(o_ref)
        pb_ref[...] = jnp.zeros_like(pb_ref)

        def im2col_store(p, slab):
            pp = jnp.roll(slab, 1, axis=0)
            pm = jnp.roll(slab, -1, axis=0)
            for kh in range(3):
                off = MRG + kh * WP
                for kw, src in ((0, pp), (1, slab), (2, pm)):
                    t = 3 * kh + kw
                    im_ref[pl.ds(p * Pp, Pp), t * C:(t + 1) * C] = \
                        jax.lax.slice_in_dim(src, off, off + Pp,
                                             axis=0).astype(BF)

        def seed_body(p, _):
            slab = x_ref[0, 0, pl.ds(p * Pp, Pp + 64), :].astype(jnp.float32)
            im2col_store(p, slab)
            return 0

        jax.lax.fori_loop(0, D2, seed_body, 0)

        nd = 5 if D % 5 == 0 else (4 if D % 4 == 0 else 2)
        MC = nd * Pp
        nt = D // nd

        io = jax.lax.broadcasted_iota(jnp.int32, (MC, 1), 0)
        rr = (io + WP) % Pp
        maskf = ((rr >= WP) & (rr < Pp - WP)
                 & (io % WP < W)).astype(jnp.float32)

        gi = jax.lax.broadcasted_iota(jnp.int32, (C, C), 0) // CG
        gj = jax.lax.broadcasted_iota(jnp.int32, (C, C), 1) // CG
        agg = (gi == gj).astype(jnp.float32)

        def conv_chunk(wref, l3, d0):
            acc = None
            for kd in range(3):
                lhs = im_ref[pl.ds((d0 + kd) * Pp, MC), :]
                wk = wref[0, l3, kd] if l3 is not None else wref[0, kd]
                pp = jnp.dot(lhs, wk, preferred_element_type=jnp.float32)
                acc = pp if acc is None else acc + pp
            return acc

        for l in range(NL):
            def body(t, carry, l=l):
                s_c, q_c = carry
                d0 = t * nd
                acc = (conv_chunk(w_ref, l, d0) + b_ref[0, l]) * maskf
                pb_ref[pl.ds(ob + d0 * Pp, MC), :] = acc
                s_c = s_c + jnp.sum(acc, axis=0, keepdims=True)
                q_c = q_c + jnp.sum(acc * acc, axis=0, keepdims=True)
                return s_c, q_c

            s_c, q_c = jax.lax.fori_loop(
                0, nt, body,
                (jnp.zeros((1, C), jnp.float32), jnp.zeros((1, C), jnp.float32)))

            gs = jnp.dot(s_c, agg, preferred_element_type=jnp.float32) / cnt
            gq = jnp.dot(q_c, agg, preferred_element_type=jnp.float32) / cnt
            var = gq - gs * gs
            inv = jax.lax.rsqrt(var + EPS)
            a = inv * g_ref[0, l]
            bb = be_ref[0, l] - gs * inv * g_ref[0, l]

            def body2(t, _):
                off = ob + t * MC
                y = pb_ref[pl.ds(off, MC), :]
                y = jnp.maximum(y * a + bb, 0.0) * maskf
                pb_ref[pl.ds(off, MC), :] = y
                return 0

            jax.lax.fori_loop(0, nt, body2, 0)

            def body3(p, _):
                slab = pb_ref[pl.ds(p * Pp, Pp + 64), :]
                im2col_store(p, slab)
                return 0

            jax.lax.fori_loop(1, D + 1, body3, 0)

        def bodyf(t, _):
            d0 = t * nd
            acc = conv_chunk(wo_ref, None, d0) * maskf
            o_ref[0, 0, pl.ds(ob + d0 * Pp, MC), :] = acc.astype(BF)
            return 0

        jax.lax.fori_loop(0, nt, bodyf, 0)

    return kern


def _tower_call(x, w, b, g, be, wo, bo, D, W, Pp, co):
    B = x.shape[0]
    T = w.shape[0]
    D2 = D + 2
    Mx = MRG + D2 * Pp + 56
    kern = _make_tower_kernel(D, W, Pp, co)
    x_spec = pl.BlockSpec((1, 1, Mx, C), lambda bi, ti: (bi, 0, 0, 0))
    w_spec = pl.BlockSpec((1, NL, 3, 9 * C, C), lambda bi, ti: (ti, 0, 0, 0, 0))
    v_spec = pl.BlockSpec((1, NL, 1, C), lambda bi, ti: (ti, 0, 0, 0))
    wo_spec = pl.BlockSpec((1, 3, 9 * C, co), lambda bi, ti: (ti, 0, 0, 0))
    bo_spec = pl.BlockSpec((1, 1, co), lambda bi, ti: (ti, 0, 0))
    o_spec = pl.BlockSpec((1, 1, Mx, co), lambda bi, ti: (bi, ti, 0, 0))
    return pl.pallas_call(
        kern,
        grid=(B, T),
        in_specs=[x_spec, w_spec, v_spec, v_spec, v_spec, wo_spec, bo_spec],
        out_specs=o_spec,
        out_shape=jax.ShapeDtypeStruct((B, T, Mx, co), BF),
        scratch_shapes=[pltpu.VMEM((D2 * Pp, 9 * C), BF),
                        pltpu.VMEM((Mx, C), jnp.float32)],
        compiler_params=pltpu.CompilerParams(
            dimension_semantics=("parallel", "parallel"),
            vmem_limit_bytes=63 * 1024 * 1024),
    )(x, w, b, g, be, wo, bo)


def _prep_x(feat, D, W, Pp):
    B = feat.shape[0]
    x = jnp.transpose(feat, (0, 2, 3, 4, 1))
    x = jnp.pad(x, ((0, 0), (1, 1), (1, 1), (0, WP - W), (0, 0)))
    x = x.reshape(B, (D + 2) * Pp, C)
    x = jnp.pad(x, ((0, 0), (MRG, 56), (0, 0)))
    return x[:, None].astype(BF)


def _prep_w(w):
    o, i = w.shape[0], w.shape[1]
    wt = jnp.transpose(w, (2, 3, 4, 1, 0))
    return wt.reshape(3, 9 * i, o).astype(BF)


def _run_level(feat, params, D):
    W = D
    H2 = D + 2
    Pp = H2 * WP
    B = feat.shape[0]

    x = _prep_x(feat, D, W, Pp)
    pc, pr = params['cls'], params['reg']
    w = jnp.stack([jnp.stack([_prep_w(p['conv'][l][0]) for l in range(NL)])
                   for p in (pc, pr)])
    b = jnp.stack([jnp.stack([p['conv'][l][1] for l in range(NL)])
                   for p in (pc, pr)])[:, :, None, :]
    g = jnp.stack([jnp.stack([p['conv'][l][2] for l in range(NL)])
                   for p in (pc, pr)])[:, :, None, :]
    be = jnp.stack([jnp.stack([p['conv'][l][3] for l in range(NL)])
                    for p in (pc, pr)])[:, :, None, :]

    (wco, boc), (wro, bor) = pc['out'], pr['out']
    nco, nro = wco.shape[0], wro.shape[0]
    co = 32
    wo = jnp.stack([
        jnp.pad(_prep_w(wco), ((0, 0), (0, 0), (0, co - nco))),
        jnp.pad(_prep_w(wro), ((0, 0), (0, 0), (0, co - nro))),
    ])
    bo = jnp.stack([jnp.pad(boc, (0, co - nco)), jnp.pad(bor, (0, co - nro))])
    bo = bo[:, None, :]

    o = _tower_call(x, w, b, g, be, wo, bo, D, W, Pp, co)

    o = o[:, :, MRG:MRG + H2 * Pp, :].reshape(B, 2, H2, H2, WP, co)
    o = o[:, :, 1:D + 1, 1:W + 1, :W, :].astype(jnp.float32)
    cls = jnp.transpose(o[:, 0, :, :, :, :nco] + boc, (0, 4, 1, 2, 3))
    reg = jnp.transpose(o[:, 1, :, :, :, :nro] + bor, (0, 4, 1, 2, 3))
    return cls, reg


def kernel(feat0, feat1, params):
    cls0, reg0 = _run_level(feat0, params, 20)
    cls1, reg1 = _run_level(feat1, params, 10)
    return (cls0, cls1, reg0, reg1)


# speedup vs baseline: 1.1858x; 1.1858x over previous
import jax
import jax.numpy as jnp
from jax.experimental import pallas as pl
from jax.experimental.pallas import tpu as pltpu

C = 128
G = 8
CG = 16
EPS = 1e-5
WP = 24
NL = 4
BF = jnp.bfloat16


def _round8(n):
    return ((n + 7) // 8) * 8


def _make_tower_kernel(D, W, Pp, rows, co):
    D2 = D + 2
    base = WP
    cnt = float(D * W * W * CG)

    def kern(x_ref, w_ref, b_ref, g_ref, be_ref, wo_ref, bo_ref, o_ref,
             im_ref, pb_ref):
        o_ref[...] = jnp.zeros_like(o_ref)
        pb_ref[...] = jnp.zeros_like(pb_ref)

        def im2col_store(p, plane):
            pp = jnp.roll(plane, 1, axis=0)
            pm = jnp.roll(plane, -1, axis=0)
            for kh in range(3):
                off = kh * WP
                for kw, src in ((0, pp), (1, plane), (2, pm)):
                    t = 3 * kh + kw
                    im_ref[p, :, t * C:(t + 1) * C] = \
                        jax.lax.slice_in_dim(src, off, off + rows,
                                             axis=0).astype(BF)

        def seed_body(p, _):
            im2col_store(p, x_ref[0, 0, p].astype(jnp.float32))
            return 0

        jax.lax.fori_loop(0, D2, seed_body, 0)

        mask = ((jax.lax.broadcasted_iota(jnp.int32, (rows, 1), 0) % WP) < W)
        maskf = mask.astype(jnp.float32)

        gi = jax.lax.broadcasted_iota(jnp.int32, (C, C), 0) // CG
        gj = jax.lax.broadcasted_iota(jnp.int32, (C, C), 1) // CG
        agg = (gi == gj).astype(jnp.float32)

        nd = 5
        nt = D // nd
        maskc = jnp.concatenate([maskf] * nd, axis=0)

        for l in range(NL):
            bias = b_ref[0, l]

            def body(t, carry, l=l):
                s_c, q_c = carry
                d0 = t * nd
                acc = None
                for kd in range(3):
                    lhs = im_ref[pl.ds(d0 + kd, nd)].reshape(nd * rows, 9 * C)
                    pp = jnp.dot(lhs, w_ref[0, l, kd],
                                 preferred_element_type=jnp.float32)
                    acc = pp if acc is None else acc + pp
                acc = (acc + bias) * maskc
                for j in range(nd):
                    pb_ref[d0 + 1 + j, pl.ds(base, rows), :] = \
                        acc[j * rows:(j + 1) * rows]
                s_c = s_c + jnp.sum(acc, axis=0, keepdims=True)
                q_c = q_c + jnp.sum(acc * acc, axis=0, keepdims=True)
                return s_c, q_c

            s_c, q_c = jax.lax.fori_loop(
                0, nt, body,
                (jnp.zeros((1, C), jnp.float32), jnp.zeros((1, C), jnp.float32)))

            gs = jnp.dot(s_c, agg, preferred_element_type=jnp.float32) / cnt
            gq = jnp.dot(q_c, agg, preferred_element_type=jnp.float32) / cnt
            var = gq - gs * gs
            inv = jax.lax.rsqrt(var + EPS)
            a = inv * g_ref[0, l]
            bb = be_ref[0, l] - gs * inv * g_ref[0, l]

            def body2(p, _):
                y = pb_ref[p, pl.ds(base, rows), :]
                y = jnp.maximum(y * a + bb, 0.0) * maskf
                pb_ref[p, pl.ds(base, rows), :] = y
                im2col_store(p, pb_ref[p])
                return 0

            jax.lax.fori_loop(1, D + 1, body2, 0)

        biaso = bo_ref[0]

        def bodyf(t, _):
            d0 = t * nd
            acc = None
            for kd in range(3):
                lhs = im_ref[pl.ds(d0 + kd, nd)].reshape(nd * rows, 9 * C)
                pp = jnp.dot(lhs, wo_ref[0, kd],
                             preferred_element_type=jnp.float32)
                acc = pp if acc is None else acc + pp
            acc = (acc + biaso) * maskc
            for j in range(nd):
                o_ref[0, 0, d0 + 1 + j, pl.ds(base, rows), :] = \
                    acc[j * rows:(j + 1) * rows]
            return 0

        jax.lax.fori_loop(0, nt, bodyf, 0)

    return kern


def _tower_call(x, w, b, g, be, wo, bo, D, W, Pp, rows, co):
    B = x.shape[0]
    T = w.shape[0]
    D2 = D + 2
    kern = _make_tower_kernel(D, W, Pp, rows, co)
    x_spec = pl.BlockSpec((1, 1, D2, Pp, C), lambda bi, ti: (bi, 0, 0, 0, 0))
    w_spec = pl.BlockSpec((1, NL, 3, 9 * C, C), lambda bi, ti: (ti, 0, 0, 0, 0))
    v_spec = pl.BlockSpec((1, NL, 1, C), lambda bi, ti: (ti, 0, 0, 0))
    wo_spec = pl.BlockSpec((1, 3, 9 * C, co), lambda bi, ti: (ti, 0, 0, 0))
    bo_spec = pl.BlockSpec((1, 1, co), lambda bi, ti: (ti, 0, 0))
    o_spec = pl.BlockSpec((1, 1, D2, Pp, co), lambda bi, ti: (bi, ti, 0, 0, 0))
    return pl.pallas_call(
        kern,
        grid=(B, T),
        in_specs=[x_spec, w_spec, v_spec, v_spec, v_spec, wo_spec, bo_spec],
        out_specs=o_spec,
        out_shape=jax.ShapeDtypeStruct((B, T, D2, Pp, co), jnp.float32),
        scratch_shapes=[pltpu.VMEM((D2, rows, 9 * C), BF),
                        pltpu.VMEM((D2, Pp, C), jnp.float32)],
        compiler_params=pltpu.CompilerParams(
            dimension_semantics=("parallel", "parallel"),
            vmem_limit_bytes=63 * 1024 * 1024),
    )(x, w, b, g, be, wo, bo)


def _prep_x(feat, D, W, Pp):
    B = feat.shape[0]
    x = jnp.transpose(feat, (0, 2, 3, 4, 1))
    x = jnp.pad(x, ((0, 0), (1, 1), (1, 1), (0, WP - W), (0, 0)))
    x = x.reshape(B, D + 2, (D + 2) * WP, C)
    return x[:, None].astype(BF)


def _prep_w(w):
    o, i = w.shape[0], w.shape[1]
    wt = jnp.transpose(w.reshape(o, i, 27), (2, 1, 0))
    return wt.reshape(3, 9 * i, o).astype(BF)


def _run_level(feat, params, D):
    W = D
    H2 = D + 2
    Pp = H2 * WP
    rows = _round8((D - 1) * WP + W)
    B = feat.shape[0]

    x = _prep_x(feat, D, W, Pp)
    pc, pr = params['cls'], params['reg']
    w = jnp.stack([jnp.stack([_prep_w(p['conv'][l][0]) for l in range(NL)])
                   for p in (pc, pr)])
    b = jnp.stack([jnp.stack([p['conv'][l][1] for l in range(NL)])
                   for p in (pc, pr)])[:, :, None, :]
    g = jnp.stack([jnp.stack([p['conv'][l][2] for l in range(NL)])
                   for p in (pc, pr)])[:, :, None, :]
    be = jnp.stack([jnp.stack([p['conv'][l][3] for l in range(NL)])
                    for p in (pc, pr)])[:, :, None, :]

    (wco, boc), (wro, bor) = pc['out'], pr['out']
    nco, nro = wco.shape[0], wro.shape[0]
    co = 32
    wo = jnp.stack([
        jnp.pad(_prep_w(wco), ((0, 0), (0, 0), (0, co - nco))),
        jnp.pad(_prep_w(wro), ((0, 0), (0, 0), (0, co - nro))),
    ])
    bo = jnp.stack([jnp.pad(boc, (0, co - nco)), jnp.pad(bor, (0, co - nro))])
    bo = bo[:, None, :]

    o = _tower_call(x, w, b, g, be, wo, bo, D, W, Pp, rows, co)

    o = o[:, :, 1:D + 1, :, :].reshape(B, 2, D, H2, WP, co)
    o = o[:, :, :, 1:W + 1, :W, :]
    cls = jnp.transpose(o[:, 0, :, :, :, :nco], (0, 4, 1, 2, 3))
    reg = jnp.transpose(o[:, 1, :, :, :, :nro], (0, 4, 1, 2, 3))
    return cls, reg


def kernel(feat0, feat1, params):
    cls0, reg0 = _run_level(feat0, params, 20)
    cls1, reg1 = _run_level(feat1, params, 10)
    return (cls0, cls1, reg0, reg1)
